# baseline (speedup 1.0000x reference)
import jax
import jax.numpy as jnp
from jax.experimental import pallas as pl
from jax.experimental.pallas import tpu as pltpu

_BM = 1024
_BK = 512


def _router_block(x_ref, w_ref, o_ref, acc_ref):
    k = pl.program_id(1)
    nk = pl.num_programs(1)
    x = x_ref[...].astype(jnp.bfloat16)
    w = w_ref[...].astype(jnp.bfloat16)
    part = jax.lax.dot_general(
        x, w,
        dimension_numbers=(((1,), (1,)), ((), ())),
        preferred_element_type=jnp.float32,
    )

    @pl.when(k == 0)
    def _():
        acc_ref[...] = part

    @pl.when(k != 0)
    def _():
        acc_ref[...] += part

    @pl.when(k == nk - 1)
    def _():
        logits = acc_ref[...]
        m = jnp.max(logits, axis=-1, keepdims=True)
        e = jnp.exp(logits - m)
        o_ref[...] = e / jnp.sum(e, axis=-1, keepdims=True)


def kernel(inputs, W):
    M, K = inputs.shape
    E = W.shape[0]
    grid = (M // _BM, K // _BK)
    return pl.pallas_call(
        _router_block,
        grid=grid,
        in_specs=[
            pl.BlockSpec((_BM, _BK), lambda i, k: (i, k)),
            pl.BlockSpec((E, _BK), lambda i, k: (0, k)),
        ],
        out_specs=pl.BlockSpec((_BM, E), lambda i, k: (i, 0)),
        out_shape=jax.ShapeDtypeStruct((M, E), jnp.float32),
        scratch_shapes=[pltpu.VMEM((_BM, E), jnp.float32)],
    )(inputs, W)


# speedup vs baseline: 1.7011x; 1.7011x over previous
import jax
import jax.numpy as jnp
from jax.experimental import pallas as pl

_BM = 1024


def _router_block(x_ref, w_ref, o_ref):
    x = x_ref[...]
    w = w_ref[...]
    logits = jax.lax.dot_general(
        x, w,
        dimension_numbers=(((1,), (1,)), ((), ())),
        preferred_element_type=jnp.float32,
    )
    m = jnp.max(logits, axis=-1, keepdims=True)
    e = jnp.exp(logits - m)
    o_ref[...] = e / jnp.sum(e, axis=-1, keepdims=True)


def kernel(inputs, W):
    M, K = inputs.shape
    E = W.shape[0]
    grid = (M // _BM,)
    return pl.pallas_call(
        _router_block,
        grid=grid,
        in_specs=[
            pl.BlockSpec((_BM, K), lambda i: (i, 0)),
            pl.BlockSpec((E, K), lambda i: (0, 0)),
        ],
        out_specs=pl.BlockSpec((_BM, E), lambda i: (i, 0)),
        out_shape=jax.ShapeDtypeStruct((M, E), jnp.float32),
    )(inputs, W)


# speedup vs baseline: 1.9936x; 1.1720x over previous
import jax
import jax.numpy as jnp
from jax.experimental import pallas as pl

_BM = 1024


def _router_block(x_ref, w_ref, o_ref):
    x = x_ref[...]
    w = w_ref[...]
    logits_t = jax.lax.dot_general(
        w, x,
        dimension_numbers=(((1,), (1,)), ((), ())),
        preferred_element_type=jnp.float32,
    )
    m = jnp.max(logits_t, axis=0, keepdims=True)
    e = jnp.exp(logits_t - m)
    o_ref[...] = e / jnp.sum(e, axis=0, keepdims=True)


def kernel(inputs, W):
    M, K = inputs.shape
    E = W.shape[0]
    grid = (M // _BM,)
    probs_t = pl.pallas_call(
        _router_block,
        grid=grid,
        in_specs=[
            pl.BlockSpec((_BM, K), lambda i: (i, 0)),
            pl.BlockSpec((E, K), lambda i: (0, 0)),
        ],
        out_specs=pl.BlockSpec((E, _BM), lambda i: (0, i)),
        out_shape=jax.ShapeDtypeStruct((E, M), jnp.float32),
    )(inputs, W)
    return probs_t.T
